# baseline (speedup 1.0000x reference)
import functools

import jax
import jax.numpy as jnp
from jax import lax
from jax.experimental import pallas as pl
from jax.experimental.pallas import tpu as pltpu
from jax.experimental.pallas import tpu_sc as plsc

N = 100000
D = 128
H = 64
S = 512
R = 20000
G = N // R

CH = 400
NCH = N // CH
SUB = 100
NW = 32
CPT = -(-NCH // NW)


def _gate_kernel(x_ref, w1_ref, b1_ref, w2_ref, logits_ref, m_ref, z_ref):
    i = pl.program_id(0)
    xb = x_ref[...].astype(jnp.bfloat16)
    h = jnp.dot(xb, w1_ref[...], preferred_element_type=jnp.float32)
    h = jnp.maximum(h + b1_ref[...], 0.0)
    g_col = jnp.dot(h.astype(jnp.bfloat16), w2_ref[...],
                    preferred_element_type=jnp.float32)
    g = jnp.transpose(g_col)
    logits_ref[0] = g

    @pl.when(i == 0)
    def _():
        m_ref[0, 0] = -jnp.inf
        z_ref[0, 0] = 0.0

    m_old = m_ref[0, 0]
    m_new = jnp.maximum(m_old, jnp.max(g))
    z_ref[0, 0] = z_ref[0, 0] * jnp.exp(m_old - m_new) + jnp.sum(jnp.exp(g - m_new))
    m_ref[0, 0] = m_new


def _weight_kernel(logits_ref, m_ref, z_ref, w_ref):
    w_ref[0] = jnp.exp(logits_ref[0] - m_ref[0, 0]) * (1.0 / z_ref[0, 0])


def _sc_pool(x_hbm, ids_hbm, w_hbm, zeros_hbm, out_hbm, xv, idv, wv, acc):
    cid = lax.axis_index("c")
    sid = lax.axis_index("s")
    wid = sid * 2 + cid

    @pl.when(sid == 0)
    def _():
        pltpu.sync_copy(zeros_hbm, acc)

    plsc.subcore_barrier()

    for k in range(CPT):
        ch = wid + NW * k

        @pl.when(ch < NCH)
        def _():
            row0 = ch * CH
            pltpu.sync_copy(x_hbm.at[pl.ds(row0, CH)], xv)
            pltpu.sync_copy(ids_hbm.at[pl.ds(ch * (CH // SUB), CH // SUB)], idv)
            pltpu.sync_copy(w_hbm.at[pl.ds(row0, CH)], wv)

            def grp(gi, carry):
                base16 = gi * 16
                wv16 = wv[pl.ds(base16, 16)]
                for r in range(16):
                    splat = wv16[jnp.full((16,), r, jnp.int32)]
                    row = base16 + r
                    for j in range(8):
                        xv[row, pl.ds(16 * j, 16)] = (
                            xv[row, pl.ds(16 * j, 16)] * splat)
                return carry

            lax.fori_loop(0, CH // 16, grp, 0)
            for j in range(CH // SUB):
                pltpu.sync_copy(xv.at[pl.ds(SUB * j, SUB)],
                                acc.at[idv.at[j]], add=True)

    plsc.subcore_barrier()

    @pl.when(sid == 0)
    def _():
        pltpu.sync_copy(acc, out_hbm.at[cid])


_sc_pool_call = functools.partial(
    pl.kernel,
    mesh=plsc.VectorSubcoreMesh(core_axis_name="c", subcore_axis_name="s"),
    out_type=jax.ShapeDtypeStruct((2, S, D), jnp.float32),
    scratch_types=[
        pltpu.VMEM((CH, D), jnp.float32),
        pltpu.VMEM((CH // SUB, SUB), jnp.int32),
        pltpu.VMEM((CH,), jnp.float32),
        pltpu.VMEM_SHARED((S, D), jnp.float32),
    ],
)(_sc_pool)


def kernel(x, batch, W1, b1, W2, b2):
    del b2
    b1r = b1.reshape(1, H)
    W1 = W1.astype(jnp.bfloat16)
    W2 = W2.astype(jnp.bfloat16)
    logits, m, z = pl.pallas_call(
        _gate_kernel,
        grid=(G,),
        in_specs=[
            pl.BlockSpec((R, D), lambda i: (i, 0)),
            pl.BlockSpec((D, H), lambda i: (0, 0)),
            pl.BlockSpec((1, H), lambda i: (0, 0)),
            pl.BlockSpec((H, 1), lambda i: (0, 0)),
        ],
        out_specs=[
            pl.BlockSpec((1, 1, R), lambda i: (i, 0, 0)),
            pl.BlockSpec(memory_space=pltpu.SMEM),
            pl.BlockSpec(memory_space=pltpu.SMEM),
        ],
        out_shape=[
            jax.ShapeDtypeStruct((G, 1, R), jnp.float32),
            jax.ShapeDtypeStruct((1, 1), jnp.float32),
            jax.ShapeDtypeStruct((1, 1), jnp.float32),
        ],
    )(x, W1, b1r, W2)
    w3 = pl.pallas_call(
        _weight_kernel,
        grid=(G,),
        in_specs=[
            pl.BlockSpec((1, 1, R), lambda i: (i, 0, 0)),
            pl.BlockSpec(memory_space=pltpu.SMEM),
            pl.BlockSpec(memory_space=pltpu.SMEM),
        ],
        out_specs=pl.BlockSpec((1, 1, R), lambda i: (i, 0, 0)),
        out_shape=jax.ShapeDtypeStruct((G, 1, R), jnp.float32),
    )(logits, m, z)
    w_flat = w3.reshape(N)
    ids2 = batch.astype(jnp.int32).reshape(N // SUB, SUB)
    zeros = jnp.zeros((S, D), jnp.float32)
    partials = _sc_pool_call(x, ids2, w_flat, zeros)
    return partials[0] + partials[1]


# speedup vs baseline: 3.5979x; 3.5979x over previous
import jax
import jax.numpy as jnp
from jax import lax
from jax.experimental import pallas as pl
from jax.experimental.pallas import tpu as pltpu

N = 100000
D = 128
H = 64
S = 512
SSUB = 128
R = 20000
G = N // R


def _fused_kernel(x_ref, ids_ref, ids_s_ref, w1_ref, b1_ref, w2_ref,
                  out_ref, m_ref, z_ref):
    i = pl.program_id(0)
    xv = x_ref[...]
    xb = xv.astype(jnp.bfloat16)
    h = jnp.dot(xb, w1_ref[...], preferred_element_type=jnp.float32)
    h = jnp.maximum(h + b1_ref[...], 0.0)
    g_col = jnp.dot(h.astype(jnp.bfloat16), w2_ref[...],
                    preferred_element_type=jnp.float32)
    g = jnp.transpose(g_col)

    @pl.when(i == 0)
    def _():
        m_ref[0, 0] = -jnp.inf
        z_ref[0, 0] = 0.0
        out_ref[...] = jnp.zeros_like(out_ref)

    m_old = m_ref[0, 0]
    m_new = jnp.maximum(m_old, jnp.max(g))
    m_ref[0, 0] = m_new
    scale = jnp.exp(m_old - m_new)

    @pl.when(jnp.logical_and(i > 0, scale < 1.0))
    def _():
        out_ref[...] *= scale

    e = jnp.exp(g - m_new)
    z_ref[0, 0] = z_ref[0, 0] * scale + jnp.sum(e)

    ids = ids_ref[0, 0, :]
    first = ids_s_ref[0, 0, 0]
    last = ids_s_ref[0, 0, R - 1]
    base = jnp.minimum((first // 8) * 8, S - SSUB)
    fits = (last - base) < SSUB

    @pl.when(fits)
    def _():
        shifted = ids - base
        seg = lax.broadcasted_iota(jnp.int32, (SSUB, R), 0)
        ohw = jnp.where(shifted[None, :] == seg, e, 0.0).astype(jnp.bfloat16)
        contrib = jnp.dot(ohw, xb, preferred_element_type=jnp.float32)
        out_ref[pl.ds(base, SSUB), :] += contrib

    @pl.when(jnp.logical_not(fits))
    def _():
        for c in range(S // SSUB):
            cbase = c * SSUB
            seg = lax.broadcasted_iota(jnp.int32, (SSUB, R), 0) + cbase
            ohw = jnp.where(ids[None, :] == seg, e, 0.0).astype(jnp.bfloat16)
            contrib = jnp.dot(ohw, xb, preferred_element_type=jnp.float32)
            out_ref[pl.ds(cbase, SSUB), :] += contrib

    @pl.when(i == G - 1)
    def _():
        out_ref[...] *= (1.0 / z_ref[0, 0])


def kernel(x, batch, W1, b1, W2, b2):
    del b2
    ids3 = batch.astype(jnp.int32).reshape(G, 1, R)
    b1r = b1.reshape(1, H)
    W1 = W1.astype(jnp.bfloat16)
    W2 = W2.astype(jnp.bfloat16)
    out = pl.pallas_call(
        _fused_kernel,
        grid=(G,),
        in_specs=[
            pl.BlockSpec((R, D), lambda i: (i, 0)),
            pl.BlockSpec((1, 1, R), lambda i: (i, 0, 0)),
            pl.BlockSpec((1, 1, R), lambda i: (i, 0, 0), memory_space=pltpu.SMEM),
            pl.BlockSpec((D, H), lambda i: (0, 0)),
            pl.BlockSpec((1, H), lambda i: (0, 0)),
            pl.BlockSpec((H, 1), lambda i: (0, 0)),
        ],
        out_specs=pl.BlockSpec((S, D), lambda i: (0, 0)),
        out_shape=jax.ShapeDtypeStruct((S, D), jnp.float32),
        scratch_shapes=[
            pltpu.SMEM((1, 1), jnp.float32),
            pltpu.SMEM((1, 1), jnp.float32),
        ],
    )(x, ids3, ids3, W1, b1r, W2)
    return out
